# baseline (speedup 1.0000x reference)
import jax
import jax.numpy as jnp
from jax.experimental import pallas as pl

BM = 256


def _agg_kernel(user_ref, latent_ref, weight_ref, int0_ref, int1_ref,
                entity_ref, out_ref):
    h = int0_ref.shape[0]
    ent = entity_ref[...].astype(jnp.bfloat16)
    agg0 = jnp.dot(int0_ref[...].astype(jnp.bfloat16), ent,
                   preferred_element_type=jnp.float32)
    agg1 = jnp.dot(int1_ref[...].astype(jnp.bfloat16), ent,
                   preferred_element_type=jnp.float32)
    agg = jnp.concatenate([agg0, agg1], axis=0)
    score = jnp.dot(user_ref[...], latent_ref[...].T,
                    preferred_element_type=jnp.float32)
    score = jax.nn.softmax(score, axis=1)
    gate = jnp.dot(score, weight_ref[...],
                   preferred_element_type=jnp.float32)
    out_ref[...] = agg * (1.0 + gate)


@jax.jit
def kernel(entity_emb, user_emb, latent_emb, weight, interact_mat):
    n_users, n_entities = interact_mat.shape
    channel = entity_emb.shape[1]
    nm = n_users // BM
    h = BM // 2

    return pl.pallas_call(
        _agg_kernel,
        grid=(nm,),
        in_specs=[
            pl.BlockSpec((BM, channel), lambda m: (m, 0)),
            pl.BlockSpec(latent_emb.shape, lambda m: (0, 0)),
            pl.BlockSpec(weight.shape, lambda m: (0, 0)),
            pl.BlockSpec((h, n_entities), lambda m: (2 * m, 0)),
            pl.BlockSpec((h, n_entities), lambda m: (2 * m + 1, 0)),
            pl.BlockSpec((n_entities, channel), lambda m: (0, 0)),
        ],
        out_specs=pl.BlockSpec((BM, channel), lambda m: (m, 0)),
        out_shape=jax.ShapeDtypeStruct((n_users, channel), jnp.float32),
    )(user_emb, latent_emb, weight, interact_mat, interact_mat, entity_emb)


# speedup vs baseline: 1.0201x; 1.0201x over previous
import jax
import jax.numpy as jnp
from jax.experimental import pallas as pl
from jax.experimental.pallas import tpu as pltpu

BM = 256


def _agg_kernel(user_ref, latent_ref, weight_ref, interact_ref, entity_ref,
                out_ref, ent_bf_ref):
    @pl.when(pl.program_id(0) == 0)
    def _cast_entity():
        ent_bf_ref[...] = entity_ref[...].astype(jnp.bfloat16)

    agg = jnp.dot(interact_ref[...].astype(jnp.bfloat16), ent_bf_ref[...],
                  preferred_element_type=jnp.float32)
    score = jnp.dot(user_ref[...], latent_ref[...].T,
                    preferred_element_type=jnp.float32)
    score = jax.nn.softmax(score, axis=1)
    gate = jnp.dot(score, weight_ref[...],
                   preferred_element_type=jnp.float32)
    out_ref[...] = agg * (1.0 + gate)


@jax.jit
def kernel(entity_emb, user_emb, latent_emb, weight, interact_mat):
    n_users, n_entities = interact_mat.shape
    channel = entity_emb.shape[1]
    nm = n_users // BM

    return pl.pallas_call(
        _agg_kernel,
        grid=(nm,),
        in_specs=[
            pl.BlockSpec((BM, channel), lambda m: (m, 0)),
            pl.BlockSpec(latent_emb.shape, lambda m: (0, 0)),
            pl.BlockSpec(weight.shape, lambda m: (0, 0)),
            pl.BlockSpec((BM, n_entities), lambda m: (m, 0)),
            pl.BlockSpec((n_entities, channel), lambda m: (0, 0)),
        ],
        out_specs=pl.BlockSpec((BM, channel), lambda m: (m, 0)),
        out_shape=jax.ShapeDtypeStruct((n_users, channel), jnp.float32),
        scratch_shapes=[pltpu.VMEM((n_entities, channel), jnp.bfloat16)],
    )(user_emb, latent_emb, weight, interact_mat, entity_emb)
